# Initial kernel scaffold; baseline (speedup 1.0000x reference)
import jax
import jax.numpy as jnp
from jax.experimental import pallas as pl


def kernel(x, memory, memory_age, Wq, bq, Wk, bk, Wv, bv, Wc, bc):
    raise NotImplementedError("write your pallas kernel here")


# speedup vs baseline: 2.2836x; 2.2836x over previous
import functools

import jax
import jax.numpy as jnp
from jax import lax
from jax.experimental import pallas as pl
from jax.experimental.pallas import tpu as pltpu
from jax.experimental.pallas import tpu_sc as plsc

B, M, D, OUT = 4096, 16384, 128, 1
BBLK = 128
MCHUNK = 2048
NCH = M // MCHUNK
R = C = 128
SCALE = 1.0 / float(jnp.sqrt(jnp.float32(D)))

_F32 = jnp.float32


def _dot(a, b, dims):
    return lax.dot_general(a, b, (dims, ((), ())),
                           preferred_element_type=_F32)


def _kv_body(mem_ref, wk_ref, bk_ref, wv_ref, bv_ref, k_ref, v_ref):
    m = mem_ref[...]
    k_ref[...] = _dot(m, wk_ref[...], ((1,), (0,))) + bk_ref[...]
    v_ref[...] = _dot(m, wv_ref[...], ((1,), (0,))) + bv_ref[...]


def _make_kv(interpret=False):
    blk = 2048
    return pl.pallas_call(
        _kv_body,
        grid=(M // blk,),
        in_specs=[
            pl.BlockSpec((blk, D), lambda i: (i, 0)),
            pl.BlockSpec((D, D), lambda i: (0, 0)),
            pl.BlockSpec((1, D), lambda i: (0, 0)),
            pl.BlockSpec((D, D), lambda i: (0, 0)),
            pl.BlockSpec((1, D), lambda i: (0, 0)),
        ],
        out_specs=[
            pl.BlockSpec((blk, D), lambda i: (i, 0)),
            pl.BlockSpec((blk, D), lambda i: (i, 0)),
        ],
        out_shape=[
            jax.ShapeDtypeStruct((M, D), _F32),
            jax.ShapeDtypeStruct((M, D), _F32),
        ],
        compiler_params=pltpu.CompilerParams(
            dimension_semantics=("parallel",)),
        interpret=interpret,
    )


def _attn_body(x_ref, k_ref, v_ref, wq_ref, bq_ref, wc1_ref, wc2_ref,
               bc_ref, attn_ref, logit_ref):
    x = x_ref[...]
    q = _dot(x, wq_ref[...], ((1,), (0,))) + bq_ref[...]

    m = jnp.full((BBLK, 1), -jnp.inf, dtype=_F32)
    l = jnp.zeros((BBLK, 1), dtype=_F32)
    for i in range(NCH):
        kc = k_ref[pl.ds(i * MCHUNK, MCHUNK), :]
        s = _dot(q, kc, ((1,), (1,))) * SCALE
        attn_ref[:, pl.ds(i * MCHUNK, MCHUNK)] = s
        cm = jnp.max(s, axis=1, keepdims=True)
        mn = jnp.maximum(m, cm)
        l = l * jnp.exp(m - mn) + jnp.sum(jnp.exp(s - mn), axis=1,
                                          keepdims=True)
        m = mn

    inv = 1.0 / l
    acc = jnp.zeros((BBLK, D), dtype=_F32)
    for i in range(NCH):
        s = attn_ref[:, pl.ds(i * MCHUNK, MCHUNK)]
        p = jnp.exp(s - m) * inv
        attn_ref[:, pl.ds(i * MCHUNK, MCHUNK)] = p
        acc = acc + _dot(p, v_ref[pl.ds(i * MCHUNK, MCHUNK), :],
                         ((1,), (0,)))

    logit_ref[...] = (_dot(x, wc1_ref[...], ((1,), (0,)))
                      + _dot(acc, wc2_ref[...], ((1,), (0,)))
                      + bc_ref[...])


def _make_attn(interpret=False):
    return pl.pallas_call(
        _attn_body,
        grid=(B // BBLK,),
        in_specs=[
            pl.BlockSpec((BBLK, D), lambda i: (i, 0)),
            pl.BlockSpec((M, D), lambda i: (0, 0)),
            pl.BlockSpec((M, D), lambda i: (0, 0)),
            pl.BlockSpec((D, D), lambda i: (0, 0)),
            pl.BlockSpec((1, D), lambda i: (0, 0)),
            pl.BlockSpec((D, OUT), lambda i: (0, 0)),
            pl.BlockSpec((D, OUT), lambda i: (0, 0)),
            pl.BlockSpec((1, OUT), lambda i: (0, 0)),
        ],
        out_specs=[
            pl.BlockSpec((BBLK, M), lambda i: (i, 0)),
            pl.BlockSpec((BBLK, OUT), lambda i: (i, 0)),
        ],
        out_shape=[
            jax.ShapeDtypeStruct((B, M), _F32),
            jax.ShapeDtypeStruct((B, OUT), _F32),
        ],
        compiler_params=pltpu.CompilerParams(
            dimension_semantics=("arbitrary",),
            vmem_limit_bytes=100 * 1024 * 1024),
        interpret=interpret,
    )


def _lex_gt(k1, i1, k2, i2):
    return (k1 > k2) | ((k1 == k2) & (i1 > i2))


def _sort_body(age_ref, perm_ref, nage_ref):
    age = age_ref[...]
    u = lax.bitcast_convert_type(age, jnp.int32)
    key = u ^ (lax.shift_right_arithmetic(u, 31) & jnp.int32(0x7FFFFFFF))
    orig_key = key

    r = lax.broadcasted_iota(jnp.int32, (R, C), 0)
    c = lax.broadcasted_iota(jnp.int32, (R, C), 1)
    j = r * C + c
    idx = j

    k = 2
    while k <= M:
        d = k // 2
        while d >= 1:
            if d < C:
                axis, s = 1, d
            else:
                axis, s = 0, d // C
            bit = (j & d) != 0
            asc = (j & k) == 0
            pk = jnp.where(bit, jnp.roll(key, s, axis=axis),
                           jnp.roll(key, -s, axis=axis))
            pi = jnp.where(bit, jnp.roll(idx, s, axis=axis),
                           jnp.roll(idx, -s, axis=axis))
            keep = _lex_gt(key, idx, pk, pi) == (bit == asc)
            key = jnp.where(keep, key, pk)
            idx = jnp.where(keep, idx, pi)
            d //= 2
        k *= 2

    perm_ref[...] = idx

    tk = lax.slice(key, (B // C - 1, C - 1), (B // C, C))
    ti = lax.slice(idx, (B // C - 1, C - 1), (B // C, C))
    sel = (orig_key < tk) | ((orig_key == tk) & (j <= ti))
    nage_ref[...] = jnp.where(sel, jnp.max(age) + 1.0, age)


def _make_sort(interpret=False):
    return pl.pallas_call(
        _sort_body,
        out_shape=[
            jax.ShapeDtypeStruct((R, C), jnp.int32),
            jax.ShapeDtypeStruct((R, C), _F32),
        ],
        interpret=interpret,
    )


_SC_NC, _SC_NS = 2, 16
_NW = _SC_NC * _SC_NS
_RPW = M // _NW
_CHUNK = 128


def _scatter_body(x_hbm, mem_hbm, perm_hbm, out_hbm, idx_v, buf, sem):
    wid = lax.axis_index("s") * _SC_NC + lax.axis_index("c")
    for cc in range(_RPW // _CHUNK):
        row = wid * (_RPW // _CHUNK) + cc
        base = row * _CHUNK
        pltpu.sync_copy(perm_hbm.at[row], idx_v)

        @pl.when(base < B)
        def _():
            pltpu.sync_copy(x_hbm.at[pl.ds(base, _CHUNK)], buf)

        @pl.when(base >= B)
        def _():
            pltpu.async_copy(mem_hbm.at[idx_v], buf, sem).wait()

        pltpu.async_copy(buf, out_hbm.at[idx_v], sem).wait()


def _make_scatter():
    mesh = plsc.VectorSubcoreMesh(core_axis_name="c", subcore_axis_name="s")
    return functools.partial(
        pl.kernel,
        out_type=jax.ShapeDtypeStruct((M, D), _F32),
        mesh=mesh,
        scratch_types=[
            pltpu.VMEM((_CHUNK,), jnp.int32),
            pltpu.VMEM((_CHUNK, D), _F32),
            pltpu.SemaphoreType.DMA,
        ],
    )(_scatter_body)


def kernel(x, memory, memory_age, Wq, bq, Wk, bk, Wv, bv, Wc, bc):
    kproj, vproj = _make_kv()(memory, Wk, bk.reshape(1, D),
                              Wv, bv.reshape(1, D))
    attn, logits = _make_attn()(x, kproj, vproj, Wq, bq.reshape(1, D),
                                Wc[:D], Wc[D:], bc.reshape(1, OUT))
    perm, nage = _make_sort()(memory_age.reshape(R, C))
    new_memory = _make_scatter()(x, memory, perm)
    new_age = nage.reshape(M)
    return logits, attn, new_memory, new_age


# speedup vs baseline: 2.5482x; 1.1159x over previous
import functools

import jax
import jax.numpy as jnp
from jax import lax
from jax.experimental import pallas as pl
from jax.experimental.pallas import tpu as pltpu
from jax.experimental.pallas import tpu_sc as plsc

B, M, D, OUT = 4096, 16384, 128, 1
BBLK = 128
MCHUNK = 2048
NCH = M // MCHUNK
R = C = 128
SCALE = 1.0 / float(jnp.sqrt(jnp.float32(D)))

_F32 = jnp.float32


def _dot(a, b, dims):
    return lax.dot_general(a, b, (dims, ((), ())),
                           preferred_element_type=_F32)


def _kv_body(mem_ref, wk_ref, bk_ref, wv_ref, bv_ref, k_ref, v_ref):
    m = mem_ref[...]
    k_ref[...] = (_dot(m, wk_ref[...], ((1,), (0,)))
                  + bk_ref[...]).astype(jnp.bfloat16)
    v_ref[...] = (_dot(m, wv_ref[...], ((1,), (0,)))
                  + bv_ref[...]).astype(jnp.bfloat16)


def _make_kv(interpret=False):
    blk = 2048
    return pl.pallas_call(
        _kv_body,
        grid=(M // blk,),
        in_specs=[
            pl.BlockSpec((blk, D), lambda i: (i, 0)),
            pl.BlockSpec((D, D), lambda i: (0, 0)),
            pl.BlockSpec((1, D), lambda i: (0, 0)),
            pl.BlockSpec((D, D), lambda i: (0, 0)),
            pl.BlockSpec((1, D), lambda i: (0, 0)),
        ],
        out_specs=[
            pl.BlockSpec((blk, D), lambda i: (i, 0)),
            pl.BlockSpec((blk, D), lambda i: (i, 0)),
        ],
        out_shape=[
            jax.ShapeDtypeStruct((M, D), jnp.bfloat16),
            jax.ShapeDtypeStruct((M, D), jnp.bfloat16),
        ],
        compiler_params=pltpu.CompilerParams(
            dimension_semantics=("parallel",)),
        interpret=interpret,
    )


def _attn_body(x_ref, k_ref, v_ref, wq_ref, bq_ref, wc1_ref, wc2_ref,
               bc_ref, attn_ref, logit_ref):
    x = x_ref[...]
    q = (_dot(x, wq_ref[...], ((1,), (0,)))
         + bq_ref[...]).astype(jnp.bfloat16)

    m = jnp.full((BBLK, 1), -jnp.inf, dtype=_F32)
    l = jnp.zeros((BBLK, 1), dtype=_F32)
    chunk_m = []
    for i in range(NCH):
        kc = k_ref[pl.ds(i * MCHUNK, MCHUNK), :]
        s = _dot(q, kc, ((1,), (1,))) * SCALE
        cm = jnp.max(s, axis=1, keepdims=True)
        mn = jnp.maximum(m, cm)
        p = jnp.exp(s - mn)
        attn_ref[:, pl.ds(i * MCHUNK, MCHUNK)] = p
        l = l * jnp.exp(m - mn) + jnp.sum(p, axis=1, keepdims=True)
        m = mn
        chunk_m.append(mn)

    inv = 1.0 / l
    acc = jnp.zeros((BBLK, D), dtype=_F32)
    for i in range(NCH):
        scale = jnp.exp(chunk_m[i] - m) * inv
        p = attn_ref[:, pl.ds(i * MCHUNK, MCHUNK)] * scale
        attn_ref[:, pl.ds(i * MCHUNK, MCHUNK)] = p
        acc = acc + _dot(p.astype(jnp.bfloat16),
                         v_ref[pl.ds(i * MCHUNK, MCHUNK), :],
                         ((1,), (0,)))

    logit_ref[...] = (_dot(x, wc1_ref[...], ((1,), (0,)))
                      + _dot(acc, wc2_ref[...], ((1,), (0,)))
                      + bc_ref[...])


def _make_attn(interpret=False):
    return pl.pallas_call(
        _attn_body,
        grid=(B // BBLK,),
        in_specs=[
            pl.BlockSpec((BBLK, D), lambda i: (i, 0)),
            pl.BlockSpec((M, D), lambda i: (0, 0)),
            pl.BlockSpec((M, D), lambda i: (0, 0)),
            pl.BlockSpec((D, D), lambda i: (0, 0)),
            pl.BlockSpec((1, D), lambda i: (0, 0)),
            pl.BlockSpec((D, OUT), lambda i: (0, 0)),
            pl.BlockSpec((D, OUT), lambda i: (0, 0)),
            pl.BlockSpec((1, OUT), lambda i: (0, 0)),
        ],
        out_specs=[
            pl.BlockSpec((BBLK, M), lambda i: (i, 0)),
            pl.BlockSpec((BBLK, OUT), lambda i: (i, 0)),
        ],
        out_shape=[
            jax.ShapeDtypeStruct((B, M), _F32),
            jax.ShapeDtypeStruct((B, OUT), _F32),
        ],
        compiler_params=pltpu.CompilerParams(
            dimension_semantics=("arbitrary",),
            vmem_limit_bytes=100 * 1024 * 1024),
        interpret=interpret,
    )


def _lex_gt(k1, i1, k2, i2):
    return (k1 > k2) | ((k1 == k2) & (i1 > i2))


def _sort_body(age_ref, perm_ref, nage_ref):
    age = age_ref[...]
    u = lax.bitcast_convert_type(age, jnp.int32)
    key = u ^ (lax.shift_right_arithmetic(u, 31) & jnp.int32(0x7FFFFFFF))
    orig_key = key

    r = lax.broadcasted_iota(jnp.int32, (R, C), 0)
    c = lax.broadcasted_iota(jnp.int32, (R, C), 1)
    j = r * C + c
    idx = j

    k = 2
    while k <= M:
        d = k // 2
        while d >= 1:
            if d < C:
                axis, s = 1, d
            else:
                axis, s = 0, d // C
            bit = (j & d) != 0
            asc = (j & k) == 0
            pk = jnp.where(bit, jnp.roll(key, s, axis=axis),
                           jnp.roll(key, -s, axis=axis))
            pi = jnp.where(bit, jnp.roll(idx, s, axis=axis),
                           jnp.roll(idx, -s, axis=axis))
            keep = _lex_gt(key, idx, pk, pi) == (bit == asc)
            key = jnp.where(keep, key, pk)
            idx = jnp.where(keep, idx, pi)
            d //= 2
        k *= 2

    perm_ref[...] = idx

    tk = lax.slice(key, (B // C - 1, C - 1), (B // C, C))
    ti = lax.slice(idx, (B // C - 1, C - 1), (B // C, C))
    sel = (orig_key < tk) | ((orig_key == tk) & (j <= ti))
    nage_ref[...] = jnp.where(sel, jnp.max(age) + 1.0, age)


def _make_sort(interpret=False):
    return pl.pallas_call(
        _sort_body,
        out_shape=[
            jax.ShapeDtypeStruct((R, C), jnp.int32),
            jax.ShapeDtypeStruct((R, C), _F32),
        ],
        interpret=interpret,
    )


_SC_NC, _SC_NS = 2, 16
_NW = _SC_NC * _SC_NS
_RPW = M // _NW
_CHUNK = 128


def _scatter_body(x_hbm, mem_hbm, perm_hbm, out_hbm, idx_v, buf, sem):
    wid = lax.axis_index("s") * _SC_NC + lax.axis_index("c")
    for cc in range(_RPW // _CHUNK):
        row = wid * (_RPW // _CHUNK) + cc
        base = row * _CHUNK
        pltpu.sync_copy(perm_hbm.at[row], idx_v)

        @pl.when(base < B)
        def _():
            pltpu.sync_copy(x_hbm.at[pl.ds(base, _CHUNK)], buf)

        @pl.when(base >= B)
        def _():
            pltpu.async_copy(mem_hbm.at[idx_v], buf, sem).wait()

        pltpu.async_copy(buf, out_hbm.at[idx_v], sem).wait()


def _make_scatter():
    mesh = plsc.VectorSubcoreMesh(core_axis_name="c", subcore_axis_name="s")
    return functools.partial(
        pl.kernel,
        out_type=jax.ShapeDtypeStruct((M, D), _F32),
        mesh=mesh,
        scratch_types=[
            pltpu.VMEM((_CHUNK,), jnp.int32),
            pltpu.VMEM((_CHUNK, D), _F32),
            pltpu.SemaphoreType.DMA,
        ],
    )(_scatter_body)


def kernel(x, memory, memory_age, Wq, bq, Wk, bk, Wv, bv, Wc, bc):
    kproj, vproj = _make_kv()(memory, Wk, bk.reshape(1, D),
                              Wv, bv.reshape(1, D))
    attn, logits = _make_attn()(x, kproj, vproj, Wq, bq.reshape(1, D),
                                Wc[:D], Wc[D:], bc.reshape(1, OUT))
    perm, nage = _make_sort()(memory_age.reshape(R, C))
    new_memory = _make_scatter()(x, memory, perm)
    new_age = nage.reshape(M)
    return logits, attn, new_memory, new_age


# speedup vs baseline: 2.9143x; 1.1437x over previous
import functools

import jax
import jax.numpy as jnp
from jax import lax
from jax.experimental import pallas as pl
from jax.experimental.pallas import tpu as pltpu
from jax.experimental.pallas import tpu_sc as plsc

B, M, D, OUT = 4096, 16384, 128, 1
BBLK = 256
MCHUNK = 2048
NCH = M // MCHUNK
R = C = 128
SCALE = D ** -0.5

_F32 = jnp.float32


def _dot(a, b, dims):
    return lax.dot_general(a, b, (dims, ((), ())),
                           preferred_element_type=_F32)


def _kv_body(mem_ref, wk_ref, bk_ref, wv_ref, bv_ref, k_ref, v_ref):
    m = mem_ref[...]
    k_ref[...] = (_dot(m, wk_ref[...], ((1,), (0,)))
                  + bk_ref[...]).astype(jnp.bfloat16)
    v_ref[...] = (_dot(m, wv_ref[...], ((1,), (0,)))
                  + bv_ref[...]).astype(jnp.bfloat16)


def _make_kv(interpret=False):
    blk = 2048
    return pl.pallas_call(
        _kv_body,
        grid=(M // blk,),
        in_specs=[
            pl.BlockSpec((blk, D), lambda i: (i, 0)),
            pl.BlockSpec((D, D), lambda i: (0, 0)),
            pl.BlockSpec((1, D), lambda i: (0, 0)),
            pl.BlockSpec((D, D), lambda i: (0, 0)),
            pl.BlockSpec((1, D), lambda i: (0, 0)),
        ],
        out_specs=[
            pl.BlockSpec((blk, D), lambda i: (i, 0)),
            pl.BlockSpec((blk, D), lambda i: (i, 0)),
        ],
        out_shape=[
            jax.ShapeDtypeStruct((M, D), jnp.bfloat16),
            jax.ShapeDtypeStruct((M, D), jnp.bfloat16),
        ],
        compiler_params=pltpu.CompilerParams(
            dimension_semantics=("parallel",)),
        interpret=interpret,
    )


def _attn_body(x_ref, k_ref, v_ref, wq_ref, bq_ref, wc1_ref, wc2_ref,
               bc_ref, attn_ref, logit_ref):
    x = x_ref[...]
    q = ((_dot(x, wq_ref[...], ((1,), (0,))) + bq_ref[...])
         * SCALE).astype(jnp.bfloat16)

    m = jnp.full((BBLK, 1), -jnp.inf, dtype=_F32)
    l = jnp.zeros((BBLK, 1), dtype=_F32)
    chunk_m = []
    for i in range(NCH):
        kc = k_ref[pl.ds(i * MCHUNK, MCHUNK), :]
        s = _dot(q, kc, ((1,), (1,)))
        cm = jnp.max(s, axis=1, keepdims=True)
        mn = jnp.maximum(m, cm)
        p = jnp.exp(s - mn)
        attn_ref[:, pl.ds(i * MCHUNK, MCHUNK)] = p
        l = l * jnp.exp(m - mn) + jnp.sum(p, axis=1, keepdims=True)
        m = mn
        chunk_m.append(mn)

    inv = 1.0 / l
    acc = jnp.zeros((BBLK, D), dtype=_F32)
    for i in range(NCH):
        scale = jnp.exp(chunk_m[i] - m) * inv
        p = attn_ref[:, pl.ds(i * MCHUNK, MCHUNK)] * scale
        attn_ref[:, pl.ds(i * MCHUNK, MCHUNK)] = p
        acc = acc + _dot(p.astype(jnp.bfloat16),
                         v_ref[pl.ds(i * MCHUNK, MCHUNK), :],
                         ((1,), (0,)))

    logit_ref[...] = (_dot(x, wc1_ref[...], ((1,), (0,)))
                      + _dot(acc, wc2_ref[...], ((1,), (0,)))
                      + bc_ref[...])


def _make_attn(interpret=False):
    return pl.pallas_call(
        _attn_body,
        grid=(B // BBLK,),
        in_specs=[
            pl.BlockSpec((BBLK, D), lambda i: (i, 0)),
            pl.BlockSpec((M, D), lambda i: (0, 0)),
            pl.BlockSpec((M, D), lambda i: (0, 0)),
            pl.BlockSpec((D, D), lambda i: (0, 0)),
            pl.BlockSpec((1, D), lambda i: (0, 0)),
            pl.BlockSpec((D, OUT), lambda i: (0, 0)),
            pl.BlockSpec((D, OUT), lambda i: (0, 0)),
            pl.BlockSpec((1, OUT), lambda i: (0, 0)),
        ],
        out_specs=[
            pl.BlockSpec((BBLK, M), lambda i: (i, 0)),
            pl.BlockSpec((BBLK, OUT), lambda i: (i, 0)),
        ],
        out_shape=[
            jax.ShapeDtypeStruct((B, M), _F32),
            jax.ShapeDtypeStruct((B, OUT), _F32),
        ],
        compiler_params=pltpu.CompilerParams(
            dimension_semantics=("arbitrary",),
            vmem_limit_bytes=100 * 1024 * 1024),
        interpret=interpret,
    )


def _lex_gt(k1, i1, k2, i2):
    return (k1 > k2) | ((k1 == k2) & (i1 > i2))


def _sort_body(age_ref, perm_ref, nage_ref):
    age = age_ref[...]
    u = lax.bitcast_convert_type(age, jnp.int32)
    key = u ^ (lax.shift_right_arithmetic(u, 31) & jnp.int32(0x7FFFFFFF))
    orig_key = key

    r = lax.broadcasted_iota(jnp.int32, (R, C), 0)
    c = lax.broadcasted_iota(jnp.int32, (R, C), 1)
    j = r * C + c
    idx = j

    k = 2
    while k <= M:
        d = k // 2
        while d >= 1:
            if d < C:
                axis, s = 1, d
            else:
                axis, s = 0, d // C
            bit = (j & d) != 0
            asc = (j & k) == 0
            pk = jnp.where(bit, jnp.roll(key, s, axis=axis),
                           jnp.roll(key, -s, axis=axis))
            pi = jnp.where(bit, jnp.roll(idx, s, axis=axis),
                           jnp.roll(idx, -s, axis=axis))
            keep = _lex_gt(key, idx, pk, pi) == (bit == asc)
            key = jnp.where(keep, key, pk)
            idx = jnp.where(keep, idx, pi)
            d //= 2
        k *= 2

    perm_ref[...] = idx

    tk = lax.slice(key, (B // C - 1, C - 1), (B // C, C))
    ti = lax.slice(idx, (B // C - 1, C - 1), (B // C, C))
    sel = (orig_key < tk) | ((orig_key == tk) & (j <= ti))
    nage_ref[...] = jnp.where(sel, jnp.max(age) + 1.0, age)


def _make_sort(interpret=False):
    return pl.pallas_call(
        _sort_body,
        out_shape=[
            jax.ShapeDtypeStruct((R, C), jnp.int32),
            jax.ShapeDtypeStruct((R, C), _F32),
        ],
        interpret=interpret,
    )


_SC_NC, _SC_NS = 2, 16
_NW = _SC_NC * _SC_NS
_RPW = M // _NW
_CHUNK = 128


def _scatter_body(x_hbm, mem_hbm, perm_hbm, out_hbm, idx_v, buf, sem):
    wid = lax.axis_index("s") * _SC_NC + lax.axis_index("c")
    for cc in range(_RPW // _CHUNK):
        row = wid * (_RPW // _CHUNK) + cc
        base = row * _CHUNK
        pltpu.sync_copy(perm_hbm.at[row], idx_v)

        @pl.when(base < B)
        def _():
            pltpu.sync_copy(x_hbm.at[pl.ds(base, _CHUNK)], buf)

        @pl.when(base >= B)
        def _():
            pltpu.async_copy(mem_hbm.at[idx_v], buf, sem).wait()

        pltpu.async_copy(buf, out_hbm.at[idx_v], sem).wait()


def _make_scatter():
    mesh = plsc.VectorSubcoreMesh(core_axis_name="c", subcore_axis_name="s")
    return functools.partial(
        pl.kernel,
        out_type=jax.ShapeDtypeStruct((M, D), _F32),
        mesh=mesh,
        scratch_types=[
            pltpu.VMEM((_CHUNK,), jnp.int32),
            pltpu.VMEM((_CHUNK, D), _F32),
            pltpu.SemaphoreType.DMA,
        ],
    )(_scatter_body)


def kernel(x, memory, memory_age, Wq, bq, Wk, bk, Wv, bv, Wc, bc):
    kproj, vproj = _make_kv()(memory, Wk, bk.reshape(1, D),
                              Wv, bv.reshape(1, D))
    attn, logits = _make_attn()(x, kproj, vproj, Wq, bq.reshape(1, D),
                                Wc[:D], Wc[D:], bc.reshape(1, OUT))
    perm, nage = _make_sort()(memory_age.reshape(R, C))
    new_memory = _make_scatter()(x, memory, perm)
    new_age = nage.reshape(M)
    return logits, attn, new_memory, new_age
